# baseline (speedup 1.0000x reference)
import functools

import jax
import jax.numpy as jnp
from jax import lax
from jax.experimental import pallas as pl
from jax.experimental.pallas import tpu as pltpu
from jax.experimental.pallas import tpu_sc as plsc

L = 16
R_BLK = 8
NBUF = 2


@jax.jit
def kernel(x, perm):
    N, D = x.shape
    info = plsc.get_sparse_core_info()
    NC, NS = info.num_cores, info.num_subcores
    NW = NC * NS
    assert N % NW == 0
    RW = N // NW
    NBLK = RW // R_BLK
    NJ = D // L
    assert NBLK % NBUF == 0 and NBLK >= 2 * NBUF

    mesh = plsc.VectorSubcoreMesh(core_axis_name="c", subcore_axis_name="s")

    @functools.partial(
        pl.kernel,
        out_type=jax.ShapeDtypeStruct((N, D), jnp.float32),
        mesh=mesh,
        compiler_params=pltpu.CompilerParams(
            needs_layout_passes=False, disable_bounds_checks=True),
        scratch_types=(
            [pltpu.VMEM((D,), jnp.int32)]
            + [pltpu.VMEM((R_BLK, D), jnp.float32)] * (2 * NBUF)
            + [pltpu.SemaphoreType.DMA] * (2 * NBUF)
        ),
    )
    def k(x_hbm, perm_hbm, out_hbm, perm_v, *rest):
        ins = rest[:NBUF]
        outs = rest[NBUF:2 * NBUF]
        isems = rest[2 * NBUF:3 * NBUF]
        osems = rest[3 * NBUF:]

        wid = lax.axis_index("s") * NC + lax.axis_index("c")
        base = wid * RW

        pltpu.sync_copy(perm_hbm, perm_v)

        def start_in(blk, b):
            pltpu.async_copy(
                x_hbm.at[pl.ds(base + blk * R_BLK, R_BLK)], ins[b], isems[b])

        def wait_in(b):
            pltpu.make_async_copy(
                x_hbm.at[pl.ds(base, R_BLK)], ins[b], isems[b]).wait()

        def start_out(blk, b):
            pltpu.async_copy(
                outs[b], out_hbm.at[pl.ds(base + blk * R_BLK, R_BLK)],
                osems[b])

        def wait_out(b):
            pltpu.make_async_copy(
                outs[b], out_hbm.at[pl.ds(base, R_BLK)], osems[b]).wait()

        def compute(b):
            ib = ins[b]
            ob = outs[b]

            @plsc.parallel_loop(0, NJ, unroll=4)
            def j_loop(j):
                col = perm_v[pl.ds(j * L, L)]
                for r in range(R_BLK):
                    rowv = jnp.full((L,), r, jnp.int32)
                    v = plsc.load_gather(ib, [rowv, col])
                    ob[r, pl.ds(j * L, L)] = v

        for b in range(NBUF - 1):
            start_in(b, b)

        @pl.loop(0, NBLK, step=NBUF)
        def blk_loop(blk0):
            for b in range(NBUF):
                blk = blk0 + b
                nxt = jnp.minimum(blk + NBUF - 1, NBLK - 1)
                start_in(nxt, (b + NBUF - 1) % NBUF)
                wait_in(b)

                @pl.when(blk >= NBUF)
                def _():
                    wait_out(b)

                compute(b)
                start_out(blk, b)

        for b in range(NBUF - 1):
            wait_in(b)
        for b in range(NBUF):
            wait_out(b)

    return k(x, perm)


# speedup vs baseline: 1.0239x; 1.0239x over previous
import functools

import jax
import jax.numpy as jnp
from jax import lax
from jax.experimental import pallas as pl
from jax.experimental.pallas import tpu as pltpu
from jax.experimental.pallas import tpu_sc as plsc

L = 16
R_BLK = 4
NBUF = 4


@jax.jit
def kernel(x, perm):
    N, D = x.shape
    info = plsc.get_sparse_core_info()
    NC, NS = info.num_cores, info.num_subcores
    NW = NC * NS
    assert N % NW == 0
    RW = N // NW
    NBLK = RW // R_BLK
    NJ = D // L
    assert NBLK % NBUF == 0 and NBLK >= 2 * NBUF

    mesh = plsc.VectorSubcoreMesh(core_axis_name="c", subcore_axis_name="s")

    @functools.partial(
        pl.kernel,
        out_type=jax.ShapeDtypeStruct((N, D), jnp.float32),
        mesh=mesh,
        compiler_params=pltpu.CompilerParams(
            needs_layout_passes=False, disable_bounds_checks=True),
        scratch_types=(
            [pltpu.VMEM((D,), jnp.int32)]
            + [pltpu.VMEM((R_BLK, D), jnp.float32)] * (2 * NBUF)
            + [pltpu.SemaphoreType.DMA] * (2 * NBUF)
        ),
    )
    def k(x_hbm, perm_hbm, out_hbm, perm_v, *rest):
        ins = rest[:NBUF]
        outs = rest[NBUF:2 * NBUF]
        isems = rest[2 * NBUF:3 * NBUF]
        osems = rest[3 * NBUF:]

        wid = lax.axis_index("s") * NC + lax.axis_index("c")
        base = wid * RW

        pltpu.sync_copy(perm_hbm, perm_v)

        def start_in(blk, b):
            pltpu.async_copy(
                x_hbm.at[pl.ds(base + blk * R_BLK, R_BLK)], ins[b], isems[b])

        def wait_in(b):
            pltpu.make_async_copy(
                x_hbm.at[pl.ds(base, R_BLK)], ins[b], isems[b]).wait()

        def start_out(blk, b):
            pltpu.async_copy(
                outs[b], out_hbm.at[pl.ds(base + blk * R_BLK, R_BLK)],
                osems[b])

        def wait_out(b):
            pltpu.make_async_copy(
                outs[b], out_hbm.at[pl.ds(base, R_BLK)], osems[b]).wait()

        def compute(b):
            ib = ins[b]
            ob = outs[b]

            @plsc.parallel_loop(0, NJ, unroll=4)
            def j_loop(j):
                col = perm_v[pl.ds(j * L, L)]
                for r in range(R_BLK):
                    rowv = jnp.full((L,), r, jnp.int32)
                    v = plsc.load_gather(ib, [rowv, col])
                    ob[r, pl.ds(j * L, L)] = v

        for b in range(NBUF - 1):
            start_in(b, b)

        @pl.loop(0, NBLK, step=NBUF)
        def blk_loop(blk0):
            for b in range(NBUF):
                blk = blk0 + b
                nxt = jnp.minimum(blk + NBUF - 1, NBLK - 1)
                start_in(nxt, (b + NBUF - 1) % NBUF)
                wait_in(b)

                @pl.when(blk >= NBUF)
                def _():
                    wait_out(b)

                compute(b)
                start_out(blk, b)

        for b in range(NBUF - 1):
            wait_in(b)
        for b in range(NBUF):
            wait_out(b)

    return k(x, perm)
